# baseline (speedup 1.0000x reference)
import jax
import jax.numpy as jnp
from jax.experimental import pallas as pl

_BATCH_BLOCK = 256


def _add_kernel(x_ref, pos_ref, out_ref):
    out_ref[...] = x_ref[...] + pos_ref[...]


def kernel(x, pos_embedding):
    batch, seq_len, embed_dim = x.shape
    flat = seq_len * embed_dim
    x2 = x.reshape(batch, flat)
    pos2 = pos_embedding.reshape(1, flat)
    bb = _BATCH_BLOCK
    grid = (batch // bb,)
    out = pl.pallas_call(
        _add_kernel,
        grid=grid,
        in_specs=[
            pl.BlockSpec((bb, flat), lambda i: (i, 0)),
            pl.BlockSpec((1, flat), lambda i: (0, 0)),
        ],
        out_specs=pl.BlockSpec((bb, flat), lambda i: (i, 0)),
        out_shape=jax.ShapeDtypeStruct((batch, flat), x.dtype),
    )(x2, pos2)
    return out.reshape(batch, seq_len, embed_dim)
